# baseline (speedup 1.0000x reference)
import functools
import random

import jax
import jax.numpy as jnp
import numpy as np
from jax import lax
from jax.experimental import pallas as pl
from jax.experimental.pallas import tpu as pltpu
from jax.experimental.pallas import tpu_sc as plsc

_LANES = 16
_NUM_CORES = 2
_NUM_SUBCORES = 16
_NUM_WORKERS = _NUM_CORES * _NUM_SUBCORES
_NBUF = 4


def _distortion_indices(h, w, a_max=3, r_max=0.7):
    random.seed(0)
    cols = h
    rows = w
    center_rows = int(np.round(random.uniform(1, rows - 2)))
    center_cols = int(np.round(random.uniform(1, cols - 2)))
    radius = random.uniform(0.03 * max(rows, cols), r_max * max(rows, cols))
    choice = random.randint(0, 1)
    spect_ratio1 = 1
    spect_ratio2 = 1
    if choice == 1:
        spect_ratio1 = random.uniform(1, a_max)
    else:
        spect_ratio2 = random.uniform(1, a_max)
    cols_np = np.arange(cols)
    rows_np = np.arange(rows)
    cols_np_t = np.tile(cols_np, (rows, 1))
    cols_pow = np.power(cols_np_t - center_cols, 2)
    rows_np_t = np.tile(rows_np, (cols, 1))
    rows_pow = np.power(rows_np_t - center_rows, 2)
    dis = np.sqrt(cols_pow + rows_pow.transpose())
    judge = (spect_ratio1 * np.abs(rows_np_t - center_rows).transpose()
             + spect_ratio2 * np.abs(cols_np_t - center_cols))
    index = np.where(judge <= radius)
    index_rows = np.rint(index[0]).astype('int64')
    index_cols = np.rint(index[1]).astype('int64')
    dis_val = dis[index]
    old_i = np.floor(dis_val / radius * (index_rows - center_rows)
                     + center_rows).astype('int64')
    old_j = np.floor(dis_val / radius * (index_cols - center_cols)
                     + center_cols).astype('int64')
    return index_rows, index_cols, old_i, old_j


@functools.lru_cache(maxsize=None)
def _index_plan(h, w):
    ir, ic, oi, oj = _distortion_indices(h, w)
    lo = int(min(ir.min(), oi.min())) // 8 * 8
    hi = -(-int(max(ir.max(), oi.max()) + 1) // 8) * 8
    hi = min(hi, h)
    src = ((oi - lo) * w + oj).astype(np.int32)
    dst = ((ir - lo) * w + ic).astype(np.int32)
    k = len(src)
    kpad = ((k + _LANES - 1) // _LANES) * _LANES
    npad = kpad - k
    pad = np.arange(npad, dtype=np.int32)
    dset = set(dst.tolist())
    assert all(int(p) not in dset for p in pad) and npad <= w
    return lo, hi - lo, np.concatenate([src, pad]), np.concatenate([dst, pad])


def _sc_permute(flat, src_off, dst_off, hw, band_start, band_len):
    total = flat.shape[0]
    num_planes = total // hw
    kpad = src_off.shape[0]
    ppw = num_planes // _NUM_WORKERS
    assert num_planes % _NUM_WORKERS == 0
    nvec = kpad // _LANES
    unroll = 12 if nvec % 12 == 0 else (4 if nvec % 4 == 0 else 1)
    band_end = band_start + band_len
    wsize = ppw * hw

    segs = [(0, band_start)]
    for i in range(ppw):
        beg = i * hw + band_end
        end = min((i + 1) * hw + band_start, wsize)
        segs.append((beg, end - beg))

    mesh = plsc.VectorSubcoreMesh(
        core_axis_name="c", subcore_axis_name="s",
        num_cores=_NUM_CORES, num_subcores=_NUM_SUBCORES)

    @functools.partial(
        pl.kernel,
        out_type=jax.ShapeDtypeStruct((total,), jnp.float32),
        mesh=mesh,
        scratch_types=(
            [pltpu.VMEM((band_len,), jnp.float32)] * _NBUF + [
            pltpu.VMEM((kpad,), jnp.int32),
            pltpu.VMEM((kpad,), jnp.int32),
            pltpu.VMEM((kpad,), jnp.float32),
            pltpu.SemaphoreType.DMA((_NBUF,)),
            pltpu.SemaphoreType.DMA((_NBUF,)),
            pltpu.SemaphoreType.DMA,
        ]),
        compiler_params=pltpu.CompilerParams(needs_layout_passes=False),
    )
    def body(feat, srch, dsth, out, buf0, buf1, buf2, buf3, srcv, dstv,
             vals, in_sems, out_sems, bulk_sem):
        bufs = [buf0, buf1, buf2, buf3]
        wid = lax.axis_index("s") * _NUM_CORES + lax.axis_index("c")
        base = wid * wsize
        pltpu.sync_copy(srch, srcv)
        pltpu.sync_copy(dsth, dstv)

        def bslice(i):
            return pl.ds(base + i * hw + band_start, band_len)

        def issue_in(i):
            pltpu.async_copy(feat.at[bslice(i)], bufs[i % _NBUF],
                             in_sems.at[i % _NBUF])

        def wait_in(i):
            pltpu.make_async_copy(feat.at[bslice(i)], bufs[i % _NBUF],
                                  in_sems.at[i % _NBUF]).wait()

        def issue_out(i):
            pltpu.async_copy(bufs[i % _NBUF], out.at[bslice(i)],
                             out_sems.at[i % _NBUF])

        def wait_out(i):
            pltpu.make_async_copy(bufs[i % _NBUF], out.at[bslice(i)],
                                  out_sems.at[i % _NBUF]).wait()

        def issue_seg(j):
            off, ln = segs[j]
            pltpu.async_copy(feat.at[pl.ds(base + off, ln)],
                             out.at[pl.ds(base + off, ln)], bulk_sem)

        for s in range(min(_NBUF - 1, ppw)):
            issue_in(s)
        issue_seg(0)

        for i in range(ppw):
            s = i % _NBUF
            wait_in(i)

            @plsc.parallel_loop(0, nvec, unroll=unroll)
            def _gather(j):
                vals[pl.ds(j * _LANES, _LANES)] = plsc.load_gather(
                    bufs[s], [srcv[pl.ds(j * _LANES, _LANES)]])

            @plsc.parallel_loop(0, nvec, unroll=unroll)
            def _scatter(j):
                plsc.store_scatter(bufs[s],
                                   [dstv[pl.ds(j * _LANES, _LANES)]],
                                   vals[pl.ds(j * _LANES, _LANES)])

            issue_out(i)
            issue_seg(i + 1)
            nxt = i + _NBUF - 1
            if nxt < ppw:
                if nxt >= _NBUF:
                    wait_out(nxt - _NBUF)
                issue_in(nxt)

        for i in range(max(0, ppw - _NBUF), ppw):
            wait_out(i)
        for off, ln in segs:
            pltpu.make_async_copy(feat.at[pl.ds(base + off, ln)],
                                  out.at[pl.ds(base + off, ln)],
                                  bulk_sem).wait()

    return body(flat, src_off, dst_off)


def kernel(feature):
    b, c, h, w = feature.shape
    lo, nrows, src_np, dst_np = _index_plan(h, w)
    src_off = jnp.asarray(src_np)
    dst_off = jnp.asarray(dst_np)
    flat = feature.reshape(b * c * h * w)
    out = _sc_permute(flat, src_off, dst_off, h * w, lo * w, nrows * w)
    return out.reshape(b, c, h, w)


# speedup vs baseline: 5.4141x; 5.4141x over previous
import functools
import random

import jax
import jax.numpy as jnp
import numpy as np
from jax import lax
from jax.experimental import pallas as pl
from jax.experimental.pallas import tpu as pltpu
from jax.experimental.pallas import tpu_sc as plsc

_LANES = 16
_NUM_CORES = 2
_NUM_SUBCORES = 16
_NUM_WORKERS = _NUM_CORES * _NUM_SUBCORES
_NBUF = 4


def _distortion_indices(h, w, a_max=3, r_max=0.7):
    random.seed(0)
    cols = h
    rows = w
    center_rows = int(np.round(random.uniform(1, rows - 2)))
    center_cols = int(np.round(random.uniform(1, cols - 2)))
    radius = random.uniform(0.03 * max(rows, cols), r_max * max(rows, cols))
    choice = random.randint(0, 1)
    spect_ratio1 = 1
    spect_ratio2 = 1
    if choice == 1:
        spect_ratio1 = random.uniform(1, a_max)
    else:
        spect_ratio2 = random.uniform(1, a_max)
    cols_np = np.arange(cols)
    rows_np = np.arange(rows)
    cols_np_t = np.tile(cols_np, (rows, 1))
    cols_pow = np.power(cols_np_t - center_cols, 2)
    rows_np_t = np.tile(rows_np, (cols, 1))
    rows_pow = np.power(rows_np_t - center_rows, 2)
    dis = np.sqrt(cols_pow + rows_pow.transpose())
    judge = (spect_ratio1 * np.abs(rows_np_t - center_rows).transpose()
             + spect_ratio2 * np.abs(cols_np_t - center_cols))
    index = np.where(judge <= radius)
    index_rows = np.rint(index[0]).astype('int64')
    index_cols = np.rint(index[1]).astype('int64')
    dis_val = dis[index]
    old_i = np.floor(dis_val / radius * (index_rows - center_rows)
                     + center_rows).astype('int64')
    old_j = np.floor(dis_val / radius * (index_cols - center_cols)
                     + center_cols).astype('int64')
    return index_rows, index_cols, old_i, old_j


@functools.lru_cache(maxsize=None)
def _index_plan(h, w):
    ir, ic, oi, oj = _distortion_indices(h, w)
    lo = int(min(ir.min(), oi.min())) // 8 * 8
    hi = -(-int(max(ir.max(), oi.max()) + 1) // 8) * 8
    hi = min(hi, h)
    src = ((oi - lo) * w + oj).astype(np.int32)
    dst = ((ir - lo) * w + ic).astype(np.int32)
    k = len(src)
    kpad = ((k + _LANES - 1) // _LANES) * _LANES
    npad = kpad - k
    pad = np.arange(npad, dtype=np.int32)
    dset = set(dst.tolist())
    assert all(int(p) not in dset for p in pad) and npad <= w
    return lo, hi - lo, np.concatenate([src, pad]), np.concatenate([dst, pad])


def _sc_permute(flat, src_off, dst_off, hw, band_start, band_len):
    total = flat.shape[0]
    num_planes = total // hw
    kpad = src_off.shape[0]
    ppw = num_planes // _NUM_WORKERS
    assert num_planes % _NUM_WORKERS == 0
    nvec = kpad // _LANES
    unroll = 12 if nvec % 12 == 0 else (4 if nvec % 4 == 0 else 1)
    band_end = band_start + band_len
    wsize = ppw * hw

    pieces = []

    def _emit_bulk(beg, end):
        while beg < end:
            ln = min(band_len, end - beg)
            pieces.append((beg, ln, False))
            beg += ln

    cursor = 0
    for i in range(ppw):
        bb = i * hw + band_start
        _emit_bulk(cursor, bb)
        pieces.append((bb, band_len, True))
        cursor = bb + band_len
    _emit_bulk(cursor, wsize)

    mesh = plsc.VectorSubcoreMesh(
        core_axis_name="c", subcore_axis_name="s",
        num_cores=_NUM_CORES, num_subcores=_NUM_SUBCORES)

    @functools.partial(
        pl.kernel,
        out_type=jax.ShapeDtypeStruct((total,), jnp.float32),
        mesh=mesh,
        scratch_types=(
            [pltpu.VMEM((band_len,), jnp.float32)] * _NBUF + [
            pltpu.VMEM((kpad,), jnp.int32),
            pltpu.VMEM((kpad,), jnp.int32),
            pltpu.VMEM((kpad,), jnp.float32),
            pltpu.SemaphoreType.DMA((_NBUF,)),
            pltpu.SemaphoreType.DMA((_NBUF,)),
        ]),
        compiler_params=pltpu.CompilerParams(needs_layout_passes=False),
    )
    def body(feat, srch, dsth, out, buf0, buf1, buf2, buf3, srcv, dstv,
             vals, in_sems, out_sems):
        bufs = [buf0, buf1, buf2, buf3]
        wid = lax.axis_index("s") * _NUM_CORES + lax.axis_index("c")
        base = wid * wsize
        pltpu.sync_copy(srch, srcv)
        pltpu.sync_copy(dsth, dstv)

        def issue_in(j):
            off, ln, _ = pieces[j]
            s = j % _NBUF
            pltpu.async_copy(feat.at[pl.ds(base + off, ln)],
                             bufs[s].at[pl.ds(0, ln)], in_sems.at[s])

        def wait_in(j):
            off, ln, _ = pieces[j]
            s = j % _NBUF
            pltpu.make_async_copy(feat.at[pl.ds(base + off, ln)],
                                  bufs[s].at[pl.ds(0, ln)],
                                  in_sems.at[s]).wait()

        def issue_out(j):
            off, ln, _ = pieces[j]
            s = j % _NBUF
            pltpu.async_copy(bufs[s].at[pl.ds(0, ln)],
                             out.at[pl.ds(base + off, ln)], out_sems.at[s])

        def wait_out(j):
            off, ln, _ = pieces[j]
            s = j % _NBUF
            pltpu.make_async_copy(bufs[s].at[pl.ds(0, ln)],
                                  out.at[pl.ds(base + off, ln)],
                                  out_sems.at[s]).wait()

        npieces = len(pieces)
        for j in range(min(_NBUF - 1, npieces)):
            issue_in(j)

        for j in range(npieces):
            s = j % _NBUF
            wait_in(j)
            if pieces[j][2]:
                buf = bufs[s]

                @plsc.parallel_loop(0, nvec, unroll=unroll)
                def _gather(v):
                    vals[pl.ds(v * _LANES, _LANES)] = plsc.load_gather(
                        buf, [srcv[pl.ds(v * _LANES, _LANES)]])

                @plsc.parallel_loop(0, nvec, unroll=unroll)
                def _scatter(v):
                    plsc.store_scatter(buf,
                                       [dstv[pl.ds(v * _LANES, _LANES)]],
                                       vals[pl.ds(v * _LANES, _LANES)])

            issue_out(j)
            nxt = j + _NBUF - 1
            if nxt < npieces:
                if nxt >= _NBUF:
                    wait_out(nxt - _NBUF)
                issue_in(nxt)

        for j in range(max(0, npieces - _NBUF), npieces):
            wait_out(j)

    return body(flat, src_off, dst_off)


def kernel(feature):
    b, c, h, w = feature.shape
    lo, nrows, src_np, dst_np = _index_plan(h, w)
    src_off = jnp.asarray(src_np)
    dst_off = jnp.asarray(dst_np)
    flat = feature.reshape(b * c * h * w)
    out = _sc_permute(flat, src_off, dst_off, h * w, lo * w, nrows * w)
    return out.reshape(b, c, h, w)


# speedup vs baseline: 31.3559x; 5.7915x over previous
import functools
import random

import jax
import jax.numpy as jnp
import numpy as np
from jax import lax
from jax.experimental import pallas as pl
from jax.experimental.pallas import tpu as pltpu
from jax.experimental.pallas import tpu_sc as plsc

_NUM_CORES = 2
_NUM_SUBCORES = 16
_NUM_WORKERS = _NUM_CORES * _NUM_SUBCORES
_NBUF = 4
_PIECE = 64


def _distortion_indices(h, w, a_max=3, r_max=0.7):
    random.seed(0)
    cols = h
    rows = w
    center_rows = int(np.round(random.uniform(1, rows - 2)))
    center_cols = int(np.round(random.uniform(1, cols - 2)))
    radius = random.uniform(0.03 * max(rows, cols), r_max * max(rows, cols))
    choice = random.randint(0, 1)
    spect_ratio1 = 1
    spect_ratio2 = 1
    if choice == 1:
        spect_ratio1 = random.uniform(1, a_max)
    else:
        spect_ratio2 = random.uniform(1, a_max)
    cols_np = np.arange(cols)
    rows_np = np.arange(rows)
    cols_np_t = np.tile(cols_np, (rows, 1))
    cols_pow = np.power(cols_np_t - center_cols, 2)
    rows_np_t = np.tile(rows_np, (cols, 1))
    rows_pow = np.power(rows_np_t - center_rows, 2)
    dis = np.sqrt(cols_pow + rows_pow.transpose())
    judge = (spect_ratio1 * np.abs(rows_np_t - center_rows).transpose()
             + spect_ratio2 * np.abs(cols_np_t - center_cols))
    index = np.where(judge <= radius)
    index_rows = np.rint(index[0]).astype('int64')
    index_cols = np.rint(index[1]).astype('int64')
    dis_val = dis[index]
    old_i = np.floor(dis_val / radius * (index_rows - center_rows)
                     + center_rows).astype('int64')
    old_j = np.floor(dis_val / radius * (index_cols - center_cols)
                     + center_cols).astype('int64')
    return index_rows, index_cols, old_i, old_j


@functools.lru_cache(maxsize=None)
def _row_map(b, h, w):
    ir, ic, oi, oj = _distortion_indices(h, w)
    m = np.arange(b * h * w, dtype=np.int32)
    for bb in range(b):
        off = bb * h * w
        m[off + ir * w + ic] = (off + oi * w + oj).astype(np.int32)
    return m


def _sc_row_gather(rows, rmap):
    nrows, ch = rows.shape
    per_w = nrows // _NUM_WORKERS
    assert nrows % _NUM_WORKERS == 0 and per_w % _PIECE == 0
    npp = per_w // _PIECE
    idx3 = rmap.reshape(_NUM_WORKERS, npp, _PIECE)

    mesh = plsc.VectorSubcoreMesh(
        core_axis_name="c", subcore_axis_name="s",
        num_cores=_NUM_CORES, num_subcores=_NUM_SUBCORES)

    @functools.partial(
        pl.kernel,
        out_type=jax.ShapeDtypeStruct((nrows, ch), jnp.float32),
        mesh=mesh,
        scratch_types=(
            [pltpu.VMEM((_PIECE, ch), jnp.float32)] * _NBUF + [
            pltpu.VMEM((npp, _PIECE), jnp.int32),
            pltpu.SemaphoreType.DMA((_NBUF,)),
            pltpu.SemaphoreType.DMA((_NBUF,)),
        ]),
        compiler_params=pltpu.CompilerParams(needs_layout_passes=False),
    )
    def body(feat, idxh, out, buf0, buf1, buf2, buf3, idxv,
             in_sems, out_sems):
        bufs = [buf0, buf1, buf2, buf3]
        wid = lax.axis_index("s") * _NUM_CORES + lax.axis_index("c")
        pltpu.sync_copy(idxh.at[wid], idxv)
        row0 = wid * per_w

        def issue_in(j):
            s = j % _NBUF
            pltpu.async_copy(feat.at[idxv.at[j]], bufs[s], in_sems.at[s])

        def wait_in(j):
            s = j % _NBUF
            pltpu.make_async_copy(feat.at[idxv.at[j]], bufs[s],
                                  in_sems.at[s]).wait()

        def issue_out(j):
            s = j % _NBUF
            pltpu.async_copy(bufs[s], out.at[pl.ds(row0 + j * _PIECE, _PIECE)],
                             out_sems.at[s])

        def wait_out(j):
            s = j % _NBUF
            pltpu.make_async_copy(bufs[s],
                                  out.at[pl.ds(row0 + j * _PIECE, _PIECE)],
                                  out_sems.at[s]).wait()

        for j in range(min(_NBUF - 1, npp)):
            issue_in(j)

        for j in range(npp):
            wait_in(j)
            issue_out(j)
            nxt = j + _NBUF - 1
            if nxt < npp:
                if nxt >= _NBUF:
                    wait_out(nxt - _NBUF)
                issue_in(nxt)

        for j in range(max(0, npp - _NBUF), npp):
            wait_out(j)

    return body(rows, jnp.asarray(idx3))


def kernel(feature):
    b, c, h, w = feature.shape
    rmap = _row_map(b, h, w)
    rows = feature.transpose(0, 2, 3, 1).reshape(b * h * w, c)
    out = _sc_row_gather(rows, rmap)
    return out.reshape(b, h, w, c).transpose(0, 3, 1, 2)


# speedup vs baseline: 31.5028x; 1.0047x over previous
import functools
import random

import jax
import jax.numpy as jnp
import numpy as np
from jax import lax
from jax.experimental import pallas as pl
from jax.experimental.pallas import tpu as pltpu
from jax.experimental.pallas import tpu_sc as plsc

_NUM_CORES = 2
_NUM_SUBCORES = 16
_NUM_WORKERS = _NUM_CORES * _NUM_SUBCORES
_NBUF = 5
_PIECE = 64


def _distortion_indices(h, w, a_max=3, r_max=0.7):
    random.seed(0)
    cols = h
    rows = w
    center_rows = int(np.round(random.uniform(1, rows - 2)))
    center_cols = int(np.round(random.uniform(1, cols - 2)))
    radius = random.uniform(0.03 * max(rows, cols), r_max * max(rows, cols))
    choice = random.randint(0, 1)
    spect_ratio1 = 1
    spect_ratio2 = 1
    if choice == 1:
        spect_ratio1 = random.uniform(1, a_max)
    else:
        spect_ratio2 = random.uniform(1, a_max)
    cols_np = np.arange(cols)
    rows_np = np.arange(rows)
    cols_np_t = np.tile(cols_np, (rows, 1))
    cols_pow = np.power(cols_np_t - center_cols, 2)
    rows_np_t = np.tile(rows_np, (cols, 1))
    rows_pow = np.power(rows_np_t - center_rows, 2)
    dis = np.sqrt(cols_pow + rows_pow.transpose())
    judge = (spect_ratio1 * np.abs(rows_np_t - center_rows).transpose()
             + spect_ratio2 * np.abs(cols_np_t - center_cols))
    index = np.where(judge <= radius)
    index_rows = np.rint(index[0]).astype('int64')
    index_cols = np.rint(index[1]).astype('int64')
    dis_val = dis[index]
    old_i = np.floor(dis_val / radius * (index_rows - center_rows)
                     + center_rows).astype('int64')
    old_j = np.floor(dis_val / radius * (index_cols - center_cols)
                     + center_cols).astype('int64')
    return index_rows, index_cols, old_i, old_j


@functools.lru_cache(maxsize=None)
def _row_map(b, h, w):
    ir, ic, oi, oj = _distortion_indices(h, w)
    m = np.arange(b * h * w, dtype=np.int32)
    for bb in range(b):
        off = bb * h * w
        m[off + ir * w + ic] = (off + oi * w + oj).astype(np.int32)
    return m


def _sc_row_gather(rows, rmap):
    nrows, ch = rows.shape
    per_w = nrows // _NUM_WORKERS
    assert nrows % _NUM_WORKERS == 0 and per_w % _PIECE == 0
    npp = per_w // _PIECE
    idx3 = rmap.reshape(_NUM_WORKERS, npp, _PIECE)

    mesh = plsc.VectorSubcoreMesh(
        core_axis_name="c", subcore_axis_name="s",
        num_cores=_NUM_CORES, num_subcores=_NUM_SUBCORES)

    @functools.partial(
        pl.kernel,
        out_type=jax.ShapeDtypeStruct((nrows, ch), jnp.float32),
        mesh=mesh,
        scratch_types=(
            [pltpu.VMEM((_PIECE, ch), jnp.float32)] * _NBUF + [
            pltpu.VMEM((npp, _PIECE), jnp.int32),
            pltpu.SemaphoreType.DMA((_NBUF,)),
            pltpu.SemaphoreType.DMA((_NBUF,)),
        ]),
        compiler_params=pltpu.CompilerParams(needs_layout_passes=False),
    )
    def body(feat, idxh, out, buf0, buf1, buf2, buf3, buf4, idxv,
             in_sems, out_sems):
        bufs = [buf0, buf1, buf2, buf3, buf4]
        wid = lax.axis_index("s") * _NUM_CORES + lax.axis_index("c")
        pltpu.sync_copy(idxh.at[wid], idxv)
        row0 = wid * per_w

        def issue_in(j):
            s = j % _NBUF
            pltpu.async_copy(feat.at[idxv.at[j]], bufs[s], in_sems.at[s])

        def wait_in(j):
            s = j % _NBUF
            pltpu.make_async_copy(feat.at[idxv.at[j]], bufs[s],
                                  in_sems.at[s]).wait()

        def issue_out(j):
            s = j % _NBUF
            pltpu.async_copy(bufs[s], out.at[pl.ds(row0 + j * _PIECE, _PIECE)],
                             out_sems.at[s])

        def wait_out(j):
            s = j % _NBUF
            pltpu.make_async_copy(bufs[s],
                                  out.at[pl.ds(row0 + j * _PIECE, _PIECE)],
                                  out_sems.at[s]).wait()

        for j in range(min(_NBUF - 1, npp)):
            issue_in(j)

        for j in range(npp):
            wait_in(j)
            issue_out(j)
            nxt = j + _NBUF - 1
            if nxt < npp:
                if nxt >= _NBUF:
                    wait_out(nxt - _NBUF)
                issue_in(nxt)

        for j in range(max(0, npp - _NBUF), npp):
            wait_out(j)

    return body(rows, jnp.asarray(idx3))


def kernel(feature):
    b, c, h, w = feature.shape
    rmap = _row_map(b, h, w)
    rows = feature.transpose(0, 2, 3, 1).reshape(b * h * w, c)
    out = _sc_row_gather(rows, rmap)
    return out.reshape(b, h, w, c).transpose(0, 3, 1, 2)
